# baseline (speedup 1.0000x reference)
import functools

import jax
import jax.numpy as jnp
from jax import lax
from jax.experimental import pallas as pl
from jax.experimental.pallas import tpu as pltpu
from jax.experimental.pallas import tpu_sc as plsc

_N = 10000
_E = 320000
_D = 128
_NC = 2
_NS = 16
_NW = _NC * _NS
_R = 320
_NPAD = _NW * _R
_DUMMY = _R
_ACC_ROWS = _R + 8
_B = 128
_KCAP = 16384
_PADV = _DUMMY
_C = 2000
_NCHUNK = _E // _C

_mesh = plsc.VectorSubcoreMesh(core_axis_name="c", subcore_axis_name="s")


def _scalar(vec):
    return lax.reduce_max(vec, (0,))


@functools.partial(
    pl.kernel,
    out_type=(jax.ShapeDtypeStruct((_NW * _KCAP,), jnp.int32),
              jax.ShapeDtypeStruct((_NW * 16,), jnp.int32),
              jax.ShapeDtypeStruct((_NPAD * 16,), jnp.float32)),
    mesh=_mesh,
    compiler_params=pltpu.CompilerParams(needs_layout_passes=False),
    scratch_types=[
        pltpu.VMEM((_C,), jnp.int32),
        pltpu.VMEM((_C,), jnp.int32),
        pltpu.VMEM((_KCAP + 16,), jnp.int32),
        pltpu.VMEM((_ACC_ROWS * 16,), jnp.float32),
        pltpu.VMEM((16,), jnp.int32),
    ],
)
def _bucket_kernel(src_hbm, dst_hbm, epack_hbm, ecnt_hbm, invdeg_hbm,
                   sbuf, dbuf, kbuf, dacc, cntb):
    c = lax.axis_index("c")
    s = lax.axis_index("s")
    wid = c * _NS + s
    lo = wid * _R

    def _prefill(i, _):
        kbuf[pl.ds(i * 16, 16)] = jnp.full((16,), _PADV, jnp.int32)
        return 0

    lax.fori_loop(0, (_KCAP + 16) // 16, _prefill, 0)

    def _zero_dacc(i, _):
        dacc[pl.ds(i * 16, 16)] = jnp.zeros((16,), jnp.float32)
        return 0

    lax.fori_loop(0, _ACC_ROWS, _zero_dacc, 0)

    lanes = lax.iota(jnp.int32, 16)
    onev = jnp.full((16,), 1.0, jnp.float32)

    def _chunk(k, cnt):
        off = pl.multiple_of(k * _C, 8)
        pltpu.sync_copy(src_hbm.at[pl.ds(off, _C)], sbuf)
        pltpu.sync_copy(dst_hbm.at[pl.ds(off, _C)], dbuf)

        def _vec(j, cnt):
            dv = dbuf[pl.ds(j * 16, 16)]
            sv = sbuf[pl.ds(j * 16, 16)]
            m = (dv >= lo) & (dv < lo + _R)
            cs = plsc.cumsum(m.astype(jnp.int32))
            pos = jnp.where(m, cnt + cs - 1, _KCAP + lanes)
            pk = jnp.left_shift(sv, 9) | (dv - lo)
            plsc.store_scatter(kbuf, [pos], pk)
            return jnp.minimum(cnt + cs[15], _KCAP - 16)

        return lax.fori_loop(0, _C // 16, _vec, cnt)

    cnt = lax.fori_loop(0, _NCHUNK, _chunk, jnp.int32(0))
    cnt_pad = ((cnt + 4 * _B - 1) // (4 * _B)) * (4 * _B)

    def _deg(i, _):
        dl16 = (kbuf[pl.ds(i * 16, 16)] & 511) * 16
        for j16 in range(16):
            r16 = dl16[j16]
            dacc[pl.ds(r16, 16)] = dacc[pl.ds(r16, 16)] + onev
        return 0

    lax.fori_loop(0, cnt_pad // 16, _deg, 0)

    def _inv(i, _):
        d = dacc[pl.ds(i * 16, 16)]
        dacc[pl.ds(i * 16, 16)] = 0.5 / jnp.maximum(d, 1.0)
        return 0

    lax.fori_loop(0, _R, _inv, 0)

    pltpu.sync_copy(dacc.at[pl.ds(0, _R * 16)],
                    invdeg_hbm.at[pl.ds(lo * 16, _R * 16)])
    pltpu.sync_copy(kbuf.at[pl.ds(0, _KCAP)],
                    epack_hbm.at[pl.ds(wid * _KCAP, _KCAP)])
    cntb[pl.ds(0, 16)] = jnp.full((16,), cnt_pad, jnp.int32)
    pltpu.sync_copy(cntb, ecnt_hbm.at[pl.ds(wid * 16, 16)])


@functools.partial(
    pl.kernel,
    out_type=jax.ShapeDtypeStruct((_NPAD, _D), jnp.float32),
    mesh=_mesh,
    compiler_params=pltpu.CompilerParams(needs_layout_passes=False),
    scratch_types=[
        pltpu.VMEM((_B,), jnp.int32),
        pltpu.VMEM((_B,), jnp.int32),
        pltpu.VMEM((_B,), jnp.int32),
        pltpu.VMEM((_B,), jnp.int32),
        pltpu.VMEM((_B,), jnp.int32),
        pltpu.VMEM((_B,), jnp.int32),
        pltpu.VMEM((_B,), jnp.int32),
        pltpu.VMEM((_B,), jnp.int32),
        pltpu.VMEM((_B, _D), jnp.float32),
        pltpu.VMEM((_B, _D), jnp.float32),
        pltpu.VMEM((_B, _D), jnp.float32),
        pltpu.VMEM((_B, _D), jnp.float32),
        pltpu.VMEM((_ACC_ROWS * _D,), jnp.float32),
        pltpu.VMEM((64, _D), jnp.float32),
        pltpu.VMEM((_R * 16,), jnp.float32),
        pltpu.VMEM((_NW * 16,), jnp.int32),
        pltpu.SemaphoreType.DMA,
        pltpu.SemaphoreType.DMA,
        pltpu.SemaphoreType.DMA,
        pltpu.SemaphoreType.DMA,
    ],
)
def _layer_kernel(x_hbm, epack_hbm, ecnt_hbm, invdeg_hbm, out_hbm,
                  ptmpA, ptmpB, ptmpC, ptmpD,
                  srcidxA, srcidxB, srcidxC, srcidxD,
                  rowsA, rowsB, rowsC, rowsD,
                  acc, xv, hv, cntb, semA, semB, semC, semD):
    c = lax.axis_index("c")
    s = lax.axis_index("s")
    wid = c * _NS + s
    base_g = wid * _R

    def _zero_acc(t, _):
        acc[pl.ds(t * 16, 16)] = jnp.zeros((16,), jnp.float32)
        return 0

    lax.fori_loop(0, _ACC_ROWS * (_D // 16), _zero_acc, 0)

    pltpu.sync_copy(ecnt_hbm, cntb)
    cnt_pad = _scalar(cntb[pl.ds(wid * 16, 16)])
    pltpu.sync_copy(invdeg_hbm.at[pl.ds(base_g * 16, _R * 16)], hv)

    def _start(ptmp, srcidx, rows, sem, off):
        pltpu.sync_copy(epack_hbm.at[pl.ds(off, _B)], ptmp)

        def _unpack(j, _):
            pk = ptmp[pl.ds(j * 16, 16)]
            srcidx[pl.ds(j * 16, 16)] = jnp.right_shift(pk, 9)
            return 0

        lax.fori_loop(0, _B // 16, _unpack, 0)
        return pltpu.async_copy(x_hbm.at[srcidx], rows, sem)

    def _accum_batch(ptmp, rows):
        def _accum(e, _):
            dl = (ptmp[pl.ds(e * 16, 16)] & 511) * _D
            for j16 in range(16):
                rb = dl[j16]
                for f in range(_D // 16):
                    acc[pl.ds(rb + f * 16, 16)] = (
                        acc[pl.ds(rb + f * 16, 16)]
                        + rows[e * 16 + j16, pl.ds(f * 16, 16)])
            return 0

        lax.fori_loop(0, _B // 16, _accum, 0)

    def _quad(q, _):
        off = pl.multiple_of(wid * _KCAP + q * 4 * _B, 8)
        hA = _start(ptmpA, srcidxA, rowsA, semA, off)
        hB = _start(ptmpB, srcidxB, rowsB, semB, off + _B)
        hC = _start(ptmpC, srcidxC, rowsC, semC, off + 2 * _B)
        hD = _start(ptmpD, srcidxD, rowsD, semD, off + 3 * _B)
        hA.wait()
        _accum_batch(ptmpA, rowsA)
        hB.wait()
        _accum_batch(ptmpB, rowsB)
        hC.wait()
        _accum_batch(ptmpC, rowsC)
        hD.wait()
        _accum_batch(ptmpD, rowsD)
        return 0

    lax.fori_loop(0, cnt_pad // (4 * _B), _quad, 0)

    for t in range(_R // 64):
        r0 = base_g + t * 64
        pltpu.sync_copy(x_hbm.at[pl.ds(r0, 64)], xv)

        def _comb(i, _, t=t):
            hb = hv[pl.ds((t * 64 + i) * 16, 16)]
            for f in range(_D // 16):
                xs = xv[i, pl.ds(f * 16, 16)]
                av = acc[pl.ds((t * 64 + i) * _D + f * 16, 16)]
                xv[i, pl.ds(f * 16, 16)] = xs * 0.5 + av * hb
            return 0

        lax.fori_loop(0, 64, _comb, 0)
        pltpu.sync_copy(xv, out_hbm.at[pl.ds(r0, 64)])


def kernel(x, edge_index):
    src = edge_index[0]
    dst = edge_index[1]
    xp = jnp.zeros((_NPAD, _D), jnp.float32).at[:_N].set(x)
    epack, ecnt, invdeg = _bucket_kernel(src, dst)

    def _step(xc, _):
        xn = _layer_kernel(xc, epack, ecnt, invdeg)
        return xn, xn

    _, ys = lax.scan(_step, xp, None, length=3)
    return jnp.concatenate([ys[0, :_N], ys[1, :_N], ys[2, :_N]], axis=-1)


# speedup vs baseline: 1.2015x; 1.2015x over previous
import functools

import jax
import jax.numpy as jnp
from jax import lax
from jax.experimental import pallas as pl
from jax.experimental.pallas import tpu as pltpu
from jax.experimental.pallas import tpu_sc as plsc

_N = 10000
_E = 320000
_D = 128
_NC = 2
_NS = 16
_NW = _NC * _NS
_R = 320
_NPAD = _NW * _R
_DUMMY = _R
_ACC_ROWS = _R + 8
_B = 128
_KCAP = 16384
_PADV = _DUMMY
_C = 4000
_NCHUNK = _E // _C

_mesh = plsc.VectorSubcoreMesh(core_axis_name="c", subcore_axis_name="s")


def _scalar(vec):
    return lax.reduce_max(vec, (0,))


@functools.partial(
    pl.kernel,
    out_type=(jax.ShapeDtypeStruct((_NW * _KCAP,), jnp.int32),
              jax.ShapeDtypeStruct((_NW * 16,), jnp.int32),
              jax.ShapeDtypeStruct((_NPAD * 16,), jnp.float32)),
    mesh=_mesh,
    compiler_params=pltpu.CompilerParams(needs_layout_passes=False),
    scratch_types=[
        pltpu.VMEM((_C,), jnp.int32),
        pltpu.VMEM((_C,), jnp.int32),
        pltpu.VMEM((_C,), jnp.int32),
        pltpu.VMEM((_C,), jnp.int32),
        pltpu.VMEM((_KCAP + 16,), jnp.int32),
        pltpu.VMEM((_ACC_ROWS * 16,), jnp.float32),
        pltpu.VMEM((16,), jnp.int32),
        pltpu.SemaphoreType.DMA,
        pltpu.SemaphoreType.DMA,
        pltpu.SemaphoreType.DMA,
        pltpu.SemaphoreType.DMA,
    ],
)
def _bucket_kernel(src_hbm, dst_hbm, epack_hbm, ecnt_hbm, invdeg_hbm,
                   sbufA, dbufA, sbufB, dbufB, kbuf, dacc, cntb,
                   semSA, semDA, semSB, semDB):
    c = lax.axis_index("c")
    s = lax.axis_index("s")
    wid = c * _NS + s
    lo = wid * _R

    def _prefill(i, _):
        kbuf[pl.ds(i * 16, 16)] = jnp.full((16,), _PADV, jnp.int32)
        return 0

    lax.fori_loop(0, (_KCAP + 16) // 16, _prefill, 0)

    def _zero_dacc(i, _):
        dacc[pl.ds(i * 16, 16)] = jnp.zeros((16,), jnp.float32)
        return 0

    lax.fori_loop(0, _ACC_ROWS, _zero_dacc, 0)

    lanes = lax.iota(jnp.int32, 16)
    onev = jnp.full((16,), 1.0, jnp.float32)

    def _process(sbuf, dbuf, cnt):
        def _vec(j, cnt):
            dv = dbuf[pl.ds(j * 16, 16)]
            sv = sbuf[pl.ds(j * 16, 16)]
            m = (dv >= lo) & (dv < lo + _R)
            cs = plsc.cumsum(m.astype(jnp.int32))
            pos = jnp.where(m, cnt + cs - 1, _KCAP + lanes)
            pk = jnp.left_shift(sv, 9) | (dv - lo)
            plsc.store_scatter(kbuf, [pos], pk)
            return jnp.minimum(cnt + cs[15], _KCAP - 16)

        return lax.fori_loop(0, _C // 16, _vec, cnt)

    def _chunk_pair(p, cnt):
        off = pl.multiple_of(p * 2 * _C, 8)
        hSA = pltpu.async_copy(src_hbm.at[pl.ds(off, _C)], sbufA, semSA)
        hDA = pltpu.async_copy(dst_hbm.at[pl.ds(off, _C)], dbufA, semDA)
        hSB = pltpu.async_copy(src_hbm.at[pl.ds(off + _C, _C)], sbufB, semSB)
        hDB = pltpu.async_copy(dst_hbm.at[pl.ds(off + _C, _C)], dbufB, semDB)
        hSA.wait()
        hDA.wait()
        cnt = _process(sbufA, dbufA, cnt)
        hSB.wait()
        hDB.wait()
        return _process(sbufB, dbufB, cnt)

    cnt = lax.fori_loop(0, _NCHUNK // 2, _chunk_pair, jnp.int32(0))
    cnt_pad = ((cnt + 2 * _B - 1) // (2 * _B)) * (2 * _B)

    def _deg(i, _):
        dl16 = (kbuf[pl.ds(i * 16, 16)] & 511) * 16
        for j16 in range(16):
            r16 = dl16[j16]
            dacc[pl.ds(r16, 16)] = dacc[pl.ds(r16, 16)] + onev
        return 0

    lax.fori_loop(0, cnt_pad // 16, _deg, 0)

    def _inv(i, _):
        d = dacc[pl.ds(i * 16, 16)]
        dacc[pl.ds(i * 16, 16)] = 0.5 / jnp.maximum(d, 1.0)
        return 0

    lax.fori_loop(0, _R, _inv, 0)

    pltpu.sync_copy(dacc.at[pl.ds(0, _R * 16)],
                    invdeg_hbm.at[pl.ds(lo * 16, _R * 16)])
    pltpu.sync_copy(kbuf.at[pl.ds(0, _KCAP)],
                    epack_hbm.at[pl.ds(wid * _KCAP, _KCAP)])
    cntb[pl.ds(0, 16)] = jnp.full((16,), cnt_pad, jnp.int32)
    pltpu.sync_copy(cntb, ecnt_hbm.at[pl.ds(wid * 16, 16)])


@functools.partial(
    pl.kernel,
    out_type=jax.ShapeDtypeStruct((_NPAD, _D), jnp.float32),
    mesh=_mesh,
    compiler_params=pltpu.CompilerParams(needs_layout_passes=False),
    scratch_types=[
        pltpu.VMEM((_B,), jnp.int32),
        pltpu.VMEM((_B,), jnp.int32),
        pltpu.VMEM((_B,), jnp.int32),
        pltpu.VMEM((_B,), jnp.int32),
        pltpu.VMEM((_B, _D), jnp.float32),
        pltpu.VMEM((_B, _D), jnp.float32),
        pltpu.VMEM((_ACC_ROWS * _D,), jnp.float32),
        pltpu.VMEM((64, _D), jnp.float32),
        pltpu.VMEM((_R * 16,), jnp.float32),
        pltpu.VMEM((_NW * 16,), jnp.int32),
        pltpu.SemaphoreType.DMA,
        pltpu.SemaphoreType.DMA,
    ],
)
def _layer_kernel(x_hbm, epack_hbm, ecnt_hbm, invdeg_hbm, out_hbm,
                  ptmpA, ptmpB, srcidxA, srcidxB, rowsA, rowsB,
                  acc, xv, hv, cntb, semA, semB):
    c = lax.axis_index("c")
    s = lax.axis_index("s")
    wid = c * _NS + s
    base_g = wid * _R

    def _zero_acc(t, _):
        acc[pl.ds(t * 16, 16)] = jnp.zeros((16,), jnp.float32)
        return 0

    lax.fori_loop(0, _ACC_ROWS * (_D // 16), _zero_acc, 0)

    pltpu.sync_copy(ecnt_hbm, cntb)
    cnt_pad = _scalar(cntb[pl.ds(wid * 16, 16)])
    pltpu.sync_copy(invdeg_hbm.at[pl.ds(base_g * 16, _R * 16)], hv)

    def _start(ptmp, srcidx, rows, sem, off):
        pltpu.sync_copy(epack_hbm.at[pl.ds(off, _B)], ptmp)

        def _unpack(j, _):
            pk = ptmp[pl.ds(j * 16, 16)]
            srcidx[pl.ds(j * 16, 16)] = jnp.right_shift(pk, 9)
            return 0

        lax.fori_loop(0, _B // 16, _unpack, 0)
        return pltpu.async_copy(x_hbm.at[srcidx], rows, sem)

    def _accum_batch(ptmp, rows):
        def _accum(e, _):
            dl = (ptmp[pl.ds(e * 16, 16)] & 511) * _D
            for j16 in range(16):
                rb = dl[j16]
                for f in range(_D // 16):
                    acc[pl.ds(rb + f * 16, 16)] = (
                        acc[pl.ds(rb + f * 16, 16)]
                        + rows[e * 16 + j16, pl.ds(f * 16, 16)])
            return 0

        lax.fori_loop(0, _B // 16, _accum, 0)

    def _pair(b, _):
        off = pl.multiple_of(wid * _KCAP + b * 2 * _B, 8)
        hA = _start(ptmpA, srcidxA, rowsA, semA, off)
        hB = _start(ptmpB, srcidxB, rowsB, semB, off + _B)
        hA.wait()
        _accum_batch(ptmpA, rowsA)
        hB.wait()
        _accum_batch(ptmpB, rowsB)
        return 0

    lax.fori_loop(0, cnt_pad // (2 * _B), _pair, 0)

    for t in range(_R // 64):
        r0 = base_g + t * 64
        pltpu.sync_copy(x_hbm.at[pl.ds(r0, 64)], xv)

        def _comb(i, _, t=t):
            hb = hv[pl.ds((t * 64 + i) * 16, 16)]
            for f in range(_D // 16):
                xs = xv[i, pl.ds(f * 16, 16)]
                av = acc[pl.ds((t * 64 + i) * _D + f * 16, 16)]
                xv[i, pl.ds(f * 16, 16)] = xs * 0.5 + av * hb
            return 0

        lax.fori_loop(0, 64, _comb, 0)
        pltpu.sync_copy(xv, out_hbm.at[pl.ds(r0, 64)])


def kernel(x, edge_index):
    src = edge_index[0]
    dst = edge_index[1]
    xp = jnp.zeros((_NPAD, _D), jnp.float32).at[:_N].set(x)
    epack, ecnt, invdeg = _bucket_kernel(src, dst)

    def _step(xc, _):
        xn = _layer_kernel(xc, epack, ecnt, invdeg)
        return xn, xn

    _, ys = lax.scan(_step, xp, None, length=3)
    return jnp.concatenate([ys[0, :_N], ys[1, :_N], ys[2, :_N]], axis=-1)
